# Initial kernel scaffold; baseline (speedup 1.0000x reference)
import jax
import jax.numpy as jnp
from jax.experimental import pallas as pl


def kernel(x, edge_index, W1, b1, W2, b2, s_w, s_b):
    raise NotImplementedError("write your pallas kernel here")


# speedup vs baseline: 19.0348x; 19.0348x over previous
import functools

import jax
import jax.numpy as jnp
from jax import lax
from jax.experimental import pallas as pl
from jax.experimental.pallas import tpu as pltpu
from jax.experimental.pallas import tpu_sc as plsc

N = 10000
FEATS = 128
HIDDEN = 128
CLASSES = 64
K = 10

NPAD = 10240
ROWS_PER_TILE = NPAD // 16
ECHUNK = 128
CPW = 82
NWORK = 32
EPAD = NWORK * CPW * ECHUNK


def _hop_body(rows_hbm, cols_hbm, z_hbm, out_hbm,
              ridx_v, cidx_v, gbuf0, gbuf1, zbuf,
              gsem0, gsem1, ssem0, ssem1, s_spmem):
    c = lax.axis_index("c")
    s = lax.axis_index("s")
    seg = s * ROWS_PER_TILE

    zero16 = jnp.zeros((16,), jnp.float32)

    def _zero_row(i, _):
        for f in range(CLASSES // 16):
            zbuf[i, pl.ds(16 * f, 16)] = zero16
        return 0

    lax.fori_loop(0, ECHUNK, _zero_row, 0)
    for kk in range(ROWS_PER_TILE // ECHUNK):
        pltpu.sync_copy(zbuf, s_spmem.at[pl.ds(seg + kk * ECHUNK, ECHUNK)])
    plsc.subcore_barrier()

    pltpu.sync_copy(rows_hbm.at[c, s], ridx_v)
    pltpu.sync_copy(cols_hbm.at[c, s], cidx_v)

    pltpu.async_copy(z_hbm.at[ridx_v.at[0]], gbuf0, gsem0)

    def _chunk_pair(i, _):
        j0 = 2 * i
        j1 = 2 * i + 1
        pltpu.make_async_copy(z_hbm.at[ridx_v.at[j0]], gbuf0, gsem0).wait()
        pltpu.async_copy(gbuf0, s_spmem.at[cidx_v.at[j0]], ssem0, add=True)
        pltpu.async_copy(z_hbm.at[ridx_v.at[j1]], gbuf1, gsem1)
        pltpu.make_async_copy(z_hbm.at[ridx_v.at[j1]], gbuf1, gsem1).wait()
        pltpu.make_async_copy(gbuf0, s_spmem.at[cidx_v.at[j0]], ssem0).wait()

        @pl.when(i + 1 < CPW // 2)
        def _():
            pltpu.async_copy(z_hbm.at[ridx_v.at[j0 + 2]], gbuf0, gsem0)

        pltpu.async_copy(gbuf1, s_spmem.at[cidx_v.at[j1]], ssem1, add=True)
        pltpu.make_async_copy(gbuf1, s_spmem.at[cidx_v.at[j1]], ssem1).wait()
        return 0

    lax.fori_loop(0, CPW // 2, _chunk_pair, 0)
    plsc.subcore_barrier()

    for kk in range(ROWS_PER_TILE // ECHUNK):
        off = seg + kk * ECHUNK
        pltpu.sync_copy(s_spmem.at[pl.ds(off, ECHUNK)], zbuf)
        pltpu.sync_copy(zbuf, out_hbm.at[c, pl.ds(off, ECHUNK)])


def _sc_hop(rows3, cols3, z):
    mesh = plsc.VectorSubcoreMesh(core_axis_name="c", subcore_axis_name="s")
    return pl.kernel(
        _hop_body,
        out_type=jax.ShapeDtypeStruct((2, NPAD, CLASSES), jnp.float32),
        mesh=mesh,
        scratch_types=[
            pltpu.VMEM((CPW, ECHUNK), jnp.int32),
            pltpu.VMEM((CPW, ECHUNK), jnp.int32),
            pltpu.VMEM((ECHUNK, CLASSES), jnp.float32),
            pltpu.VMEM((ECHUNK, CLASSES), jnp.float32),
            pltpu.VMEM((ECHUNK, CLASSES), jnp.float32),
            pltpu.SemaphoreType.DMA,
            pltpu.SemaphoreType.DMA,
            pltpu.SemaphoreType.DMA,
            pltpu.SemaphoreType.DMA,
            pltpu.VMEM_SHARED((NPAD, CLASSES), jnp.float32),
        ],
        compiler_params=pltpu.CompilerParams(use_tc_tiling_on_sc=False),
        name="dagnn_sc_hop",
    )(rows3, cols3, z)


def _mlp_body(x_ref, w1_ref, b1_ref, w2_ref, b2_ref, dinv_ref, z0_ref):
    h = jnp.maximum(
        jnp.dot(x_ref[...], w1_ref[...], preferred_element_type=jnp.float32)
        + b1_ref[...], 0.0)
    h = jnp.dot(h, w2_ref[...], preferred_element_type=jnp.float32) + b2_ref[...]
    z0_ref[...] = h * dinv_ref[...]


def _tc_mlp(x_pad, W1, b1, W2, b2, dinv2):
    bn = 512
    grid = (NPAD // bn,)
    return pl.pallas_call(
        _mlp_body,
        grid=grid,
        in_specs=[
            pl.BlockSpec((bn, FEATS), lambda i: (i, 0)),
            pl.BlockSpec((FEATS, HIDDEN), lambda i: (0, 0)),
            pl.BlockSpec((1, HIDDEN), lambda i: (0, 0)),
            pl.BlockSpec((HIDDEN, CLASSES), lambda i: (0, 0)),
            pl.BlockSpec((1, CLASSES), lambda i: (0, 0)),
            pl.BlockSpec((bn, 1), lambda i: (i, 0)),
        ],
        out_specs=pl.BlockSpec((bn, CLASSES), lambda i: (i, 0)),
        out_shape=jax.ShapeDtypeStruct((NPAD, CLASSES), jnp.float32),
    )(x_pad, W1, b1.reshape(1, HIDDEN), W2, b2.reshape(1, CLASSES), dinv2)


def _combine_body(p_ref, d2_ref, z_ref):
    z_ref[...] = d2_ref[...] * (p_ref[0] + p_ref[1])


def _tc_combine(partials, d2c):
    bn = 512
    grid = (NPAD // bn,)
    return pl.pallas_call(
        _combine_body,
        grid=grid,
        in_specs=[
            pl.BlockSpec((2, bn, CLASSES), lambda i: (0, i, 0)),
            pl.BlockSpec((bn, 1), lambda i: (i, 0)),
        ],
        out_specs=pl.BlockSpec((bn, CLASSES), lambda i: (i, 0)),
        out_shape=jax.ShapeDtypeStruct((NPAD, CLASSES), jnp.float32),
    )(partials, d2c)


def _retention_body(*refs):
    z_refs = refs[:K]
    p_ref, d2_ref, rd_ref, sw_ref, sb_ref, out_ref = refs[K:]
    rd = rd_ref[...]
    sw = sw_ref[...]
    sb = sb_ref[0, 0]
    acc = jnp.zeros_like(out_ref[...])
    curs = [zr[...] * rd for zr in z_refs]
    curs.append((p_ref[0] + p_ref[1]) * d2_ref[...] * rd)
    for cur in curs:
        r = jax.nn.sigmoid(
            jnp.dot(cur, sw, preferred_element_type=jnp.float32) + sb)
        acc = acc + r * cur
    m = jnp.max(acc, axis=1, keepdims=True)
    e = jnp.exp(acc - m)
    out_ref[...] = acc - m - jnp.log(jnp.sum(e, axis=1, keepdims=True))


def _tc_retention(zs, partials, d2c, rdc, s_w, s_b):
    bn = 512
    grid = (NPAD // bn,)
    zspec = pl.BlockSpec((bn, CLASSES), lambda i: (i, 0))
    vspec = pl.BlockSpec((bn, 1), lambda i: (i, 0))
    return pl.pallas_call(
        _retention_body,
        grid=grid,
        in_specs=[zspec] * K + [
            pl.BlockSpec((2, bn, CLASSES), lambda i: (0, i, 0)),
            vspec,
            vspec,
            pl.BlockSpec((CLASSES, 1), lambda i: (0, 0)),
            pl.BlockSpec((1, 1), lambda i: (0, 0)),
        ],
        out_specs=pl.BlockSpec((bn, CLASSES), lambda i: (i, 0)),
        out_shape=jax.ShapeDtypeStruct((NPAD, CLASSES), jnp.float32),
    )(*zs, partials, d2c, rdc, s_w.reshape(CLASSES, 1),
      s_b.reshape(1, 1))


def kernel(x, edge_index, W1, b1, W2, b2, s_w, s_b):
    loop = jnp.arange(N, dtype=jnp.int32)
    row = jnp.concatenate([edge_index[0], loop])
    col = jnp.concatenate([edge_index[1], loop])
    npe = EPAD - row.shape[0]
    pad_idx = N + (jnp.arange(npe, dtype=jnp.int32) % (NPAD - N))
    rows3 = jnp.concatenate([row, pad_idx]).reshape(2, 16, CPW, ECHUNK)
    cols3 = jnp.concatenate([col, pad_idx]).reshape(2, 16, CPW, ECHUNK)

    ones_z = jnp.ones((NPAD, CLASSES), jnp.float32)
    deg_p = _sc_hop(rows3, cols3, ones_z)
    deg = deg_p[0, :, 0] + deg_p[1, :, 0]
    dinv = jnp.where(deg > 0, lax.rsqrt(jnp.maximum(deg, 1e-12)), 0.0)
    d2c = (dinv * dinv).reshape(NPAD, 1)
    rdc = jnp.sqrt(deg).reshape(NPAD, 1)

    x_pad = jnp.pad(x, ((0, NPAD - N), (0, 0)))
    z = _tc_mlp(x_pad, W1, b1, W2, b2, dinv.reshape(NPAD, 1))

    zs = [z]
    partials = None
    for _ in range(K):
        partials = _sc_hop(rows3, cols3, zs[-1])
        if len(zs) < K:
            zs.append(_tc_combine(partials, d2c))

    out = _tc_retention(zs, partials, d2c, rdc, s_w, s_b)
    return out[:N]
